# Initial kernel scaffold; baseline (speedup 1.0000x reference)
import jax
import jax.numpy as jnp
from jax.experimental import pallas as pl


def kernel(hidden_states, Wg, W1, b1, W2, b2):
    raise NotImplementedError("write your pallas kernel here")


# speedup vs baseline: 7.9469x; 7.9469x over previous
import functools

import jax
import jax.numpy as jnp
from jax import lax
from jax.experimental import pallas as pl
from jax.experimental.pallas import tpu as pltpu
from jax.experimental.pallas import tpu_sc as plsc

_E = 64
_T = 8192
_D = 768
_FF = 768
_BM = 256
_MAXT = _T // _BM + _E - 1
_NW = 32
_NCH = 2
_CH = (_T // _NW) // _NCH


def _router_body(x_ref, wg_ref, w_ref, id_ref):
    logits = jnp.dot(x_ref[...], wg_ref[...], preferred_element_type=jnp.float32)
    m = jnp.max(logits, axis=-1, keepdims=True)
    ex = jnp.exp(logits - m)
    p = ex / jnp.sum(ex, axis=-1, keepdims=True)
    pmax = jnp.max(p, axis=-1)
    col = lax.broadcasted_iota(jnp.int32, p.shape, 1)
    idx = jnp.min(jnp.where(p >= pmax[:, None], col, p.shape[-1]), axis=-1)
    w_ref[...] = pmax
    id_ref[...] = idx


def _router(x, wg):
    return pl.pallas_call(
        _router_body,
        out_shape=(
            jax.ShapeDtypeStruct((_T,), jnp.float32),
            jax.ShapeDtypeStruct((_T,), jnp.int32),
        ),
    )(x, wg)


def _sc_gather(table, idx3):
    t_rows, d = table.shape
    nw, nch, ch = idx3.shape
    bpw = nch * ch
    mesh = plsc.VectorSubcoreMesh(core_axis_name="c", subcore_axis_name="s")

    @functools.partial(
        pl.kernel,
        mesh=mesh,
        out_type=jax.ShapeDtypeStruct((t_rows, d), jnp.float32),
        scratch_types=[
            pltpu.VMEM((nch, ch), jnp.int32),
            pltpu.VMEM((ch, d), jnp.float32),
            pltpu.SemaphoreType.DMA,
        ],
    )
    def gk(table_hbm, idx_hbm, out_hbm, idx_v, rows_v, sem):
        wid = lax.axis_index("s") * 2 + lax.axis_index("c")
        pltpu.sync_copy(idx_hbm.at[wid], idx_v)
        for j in range(nch):
            pltpu.async_copy(table_hbm.at[idx_v.at[j]], rows_v, sem).wait()
            pltpu.sync_copy(rows_v, out_hbm.at[pl.ds(wid * bpw + j * ch, ch)])

    return gk(table, idx3)


def _mlp_body(b_ref, e_ref, s_ref, t_ref,
              x_ref, w1_ref, b1_ref, w2_ref, b2_ref, ws_ref, out_ref):
    i = pl.program_id(0)
    s = s_ref[i]
    t = t_ref[i]
    base = b_ref[i] * _BM

    @pl.when(s < t)
    def _():
        x = x_ref[...]
        h = jnp.dot(x, w1_ref[0], preferred_element_type=jnp.float32)
        h = jnp.maximum(h + b1_ref[0], 0.0)
        y = jnp.dot(h, w2_ref[0], preferred_element_type=jnp.float32)
        y = y + b2_ref[0]
        y = y * ws_ref[...][:, None]
        rows = base + lax.broadcasted_iota(jnp.int32, (_BM, 1), 0)
        mask = (rows >= s) & (rows < t)
        out_ref[...] = jnp.where(mask, y, out_ref[...])


def _grouped_mlp(x_sorted, w1, b1, w2, b2, w_sorted, sched_b, sched_e, sched_s, sched_t):
    grid_spec = pltpu.PrefetchScalarGridSpec(
        num_scalar_prefetch=4,
        grid=(_MAXT,),
        in_specs=[
            pl.BlockSpec((_BM, _D), lambda i, b, e, s, t: (b[i], 0)),
            pl.BlockSpec((1, _D, _FF), lambda i, b, e, s, t: (e[i], 0, 0)),
            pl.BlockSpec((1, 1, _FF), lambda i, b, e, s, t: (e[i], 0, 0)),
            pl.BlockSpec((1, _FF, _D), lambda i, b, e, s, t: (e[i], 0, 0)),
            pl.BlockSpec((1, 1, _D), lambda i, b, e, s, t: (e[i], 0, 0)),
            pl.BlockSpec((_BM,), lambda i, b, e, s, t: (b[i],)),
        ],
        out_specs=pl.BlockSpec((_BM, _D), lambda i, b, e, s, t: (b[i], 0)),
    )
    return pl.pallas_call(
        _mlp_body,
        grid_spec=grid_spec,
        out_shape=jax.ShapeDtypeStruct((_T, _D), jnp.float32),
        compiler_params=pltpu.CompilerParams(dimension_semantics=("arbitrary",)),
    )(sched_b, sched_e, sched_s, sched_t, x_sorted, w1,
      b1.reshape(_E, 1, _FF), w2, b2.reshape(_E, 1, _D), w_sorted)


def _schedule(offsets):
    i32 = jnp.int32
    s_e = offsets[:-1]
    t_e = offsets[1:]
    nonempty = t_e > s_e
    first = s_e // _BM
    nblk = jnp.where(nonempty, (t_e - 1) // _BM - first + 1, 0)
    c = jnp.concatenate([jnp.zeros((1,), i32), jnp.cumsum(nblk).astype(i32)])
    total = c[-1]
    j = jnp.arange(_MAXT, dtype=i32)
    ej = jnp.searchsorted(c, j, side="right").astype(i32) - 1
    ej = jnp.minimum(ej, _E - 1)
    bj = first[ej] + (j - c[ej])
    valid = j < total
    jp = total - 1
    ep = jnp.searchsorted(c, jp, side="right").astype(i32) - 1
    bp = first[ep] + (jp - c[ep])
    ej = jnp.where(valid, ej, ep)
    bj = jnp.where(valid, bj, bp)
    sj = jnp.where(valid, s_e[ej], 0)
    tj = jnp.where(valid, t_e[ej], 0)
    return bj.astype(i32), ej.astype(i32), sj.astype(i32), tj.astype(i32)


def kernel(hidden_states, Wg, W1, b1, W2, b2):
    x = hidden_states
    w_tok, e_tok = _router(x, Wg)

    iota = jnp.arange(_T, dtype=jnp.int32)
    eid_sorted, perm = lax.sort((e_tok, iota), num_keys=1)
    offsets = jnp.searchsorted(
        eid_sorted, jnp.arange(_E + 1, dtype=jnp.int32), side="left"
    ).astype(jnp.int32)
    inv_perm = jnp.zeros((_T,), jnp.int32).at[perm].set(iota)
    w_sorted = w_tok[perm]
    sched_b, sched_e, sched_s, sched_t = _schedule(offsets)

    x_sorted = _sc_gather(x, perm.reshape(_NW, _NCH, _CH))
    y_sorted = _grouped_mlp(x_sorted, W1, b1, W2, b2, w_sorted,
                            sched_b, sched_e, sched_s, sched_t)
    out = _sc_gather(y_sorted, inv_perm.reshape(_NW, _NCH, _CH))
    return out


# speedup vs baseline: 8.7542x; 1.1016x over previous
import functools

import jax
import jax.numpy as jnp
from jax import lax
from jax.experimental import pallas as pl
from jax.experimental.pallas import tpu as pltpu
from jax.experimental.pallas import tpu_sc as plsc

_E = 64
_T = 8192
_D = 768
_FF = 768
_BM = 256
_MAXT = _T // _BM + _E - 1
_NW = 32
_NCH = 2
_CH = (_T // _NW) // _NCH


def _router_body(x_ref, wg_ref, w_ref, id_ref):
    logits = jnp.dot(x_ref[...], wg_ref[...], preferred_element_type=jnp.float32)
    m = jnp.max(logits, axis=-1, keepdims=True)
    ex = jnp.exp(logits - m)
    p = ex / jnp.sum(ex, axis=-1, keepdims=True)
    pmax = jnp.max(p, axis=-1)
    col = lax.broadcasted_iota(jnp.int32, p.shape, 1)
    idx = jnp.min(jnp.where(p >= pmax[:, None], col, p.shape[-1]), axis=-1)
    w_ref[...] = pmax
    id_ref[...] = idx


def _router(x, wg):
    return pl.pallas_call(
        _router_body,
        out_shape=(
            jax.ShapeDtypeStruct((_T,), jnp.float32),
            jax.ShapeDtypeStruct((_T,), jnp.int32),
        ),
    )(x, wg)


def _sc_gather(table, idx3):
    t_rows, d = table.shape
    nw, nch, ch = idx3.shape
    bpw = nch * ch
    mesh = plsc.VectorSubcoreMesh(core_axis_name="c", subcore_axis_name="s")

    @functools.partial(
        pl.kernel,
        mesh=mesh,
        out_type=jax.ShapeDtypeStruct((t_rows, d), jnp.float32),
        scratch_types=[
            pltpu.VMEM((nch, ch), jnp.int32),
            pltpu.VMEM((ch, d), jnp.float32),
            pltpu.SemaphoreType.DMA,
        ],
    )
    def gk(table_hbm, idx_hbm, out_hbm, idx_v, rows_v, sem):
        wid = lax.axis_index("s") * 2 + lax.axis_index("c")
        pltpu.sync_copy(idx_hbm.at[wid], idx_v)
        for j in range(nch):
            pltpu.async_copy(table_hbm.at[idx_v.at[j]], rows_v, sem).wait()
            pltpu.sync_copy(rows_v, out_hbm.at[pl.ds(wid * bpw + j * ch, ch)])

    return gk(table, idx3)


def _sc_scatter(rows, idx3, out_rows):
    t_rows, d = rows.shape
    nw, nch, ch = idx3.shape
    bpw = nch * ch
    mesh = plsc.VectorSubcoreMesh(core_axis_name="c", subcore_axis_name="s")

    @functools.partial(
        pl.kernel,
        mesh=mesh,
        out_type=jax.ShapeDtypeStruct((out_rows, d), jnp.float32),
        scratch_types=[
            pltpu.VMEM((nch, ch), jnp.int32),
            pltpu.VMEM((ch, d), jnp.float32),
            pltpu.SemaphoreType.DMA,
        ],
    )
    def sk(rows_hbm, idx_hbm, out_hbm, idx_v, rows_v, sem):
        wid = lax.axis_index("s") * 2 + lax.axis_index("c")
        pltpu.sync_copy(idx_hbm.at[wid], idx_v)
        for j in range(nch):
            pltpu.sync_copy(rows_hbm.at[pl.ds(wid * bpw + j * ch, ch)], rows_v)
            pltpu.async_copy(rows_v, out_hbm.at[idx_v.at[j]], sem).wait()

    return sk(rows, idx3)


def _mlp_body(b_ref, e_ref, s_ref, t_ref,
              x_ref, w1_ref, b1_ref, w2_ref, b2_ref, ws_ref, out_ref):
    i = pl.program_id(0)
    s = s_ref[i]
    t = t_ref[i]
    base = b_ref[i] * _BM

    @pl.when(s < t)
    def _():
        x = x_ref[...].astype(jnp.bfloat16)
        h = jnp.dot(x, w1_ref[0].astype(jnp.bfloat16),
                    preferred_element_type=jnp.float32)
        h = jnp.maximum(h + b1_ref[0], 0.0).astype(jnp.bfloat16)
        y = jnp.dot(h, w2_ref[0].astype(jnp.bfloat16),
                    preferred_element_type=jnp.float32)
        y = y + b2_ref[0]
        y = y * ws_ref[...][:, None]
        rows = base + lax.broadcasted_iota(jnp.int32, (_BM, 1), 0)
        mask = (rows >= s) & (rows < t)
        out_ref[...] = jnp.where(mask, y, out_ref[...])


def _grouped_mlp(x_sorted, w1, b1, w2, b2, w_sorted, sched_b, sched_e, sched_s, sched_t):
    grid_spec = pltpu.PrefetchScalarGridSpec(
        num_scalar_prefetch=4,
        grid=(_MAXT,),
        in_specs=[
            pl.BlockSpec((_BM, _D), lambda i, b, e, s, t: (b[i], 0)),
            pl.BlockSpec((1, _D, _FF), lambda i, b, e, s, t: (e[i], 0, 0)),
            pl.BlockSpec((1, 1, _FF), lambda i, b, e, s, t: (e[i], 0, 0)),
            pl.BlockSpec((1, _FF, _D), lambda i, b, e, s, t: (e[i], 0, 0)),
            pl.BlockSpec((1, 1, _D), lambda i, b, e, s, t: (e[i], 0, 0)),
            pl.BlockSpec((_BM,), lambda i, b, e, s, t: (b[i],)),
        ],
        out_specs=pl.BlockSpec((_BM, _D), lambda i, b, e, s, t: (b[i], 0)),
    )
    return pl.pallas_call(
        _mlp_body,
        grid_spec=grid_spec,
        out_shape=jax.ShapeDtypeStruct((_T, _D), jnp.float32),
        compiler_params=pltpu.CompilerParams(dimension_semantics=("arbitrary",)),
    )(sched_b, sched_e, sched_s, sched_t, x_sorted, w1,
      b1.reshape(_E, 1, _FF), w2, b2.reshape(_E, 1, _D), w_sorted)


def _schedule(offsets):
    i32 = jnp.int32
    s_e = offsets[:-1]
    t_e = offsets[1:]
    nonempty = t_e > s_e
    first = s_e // _BM
    nblk = jnp.where(nonempty, (t_e - 1) // _BM - first + 1, 0)
    c = jnp.concatenate([jnp.zeros((1,), i32), jnp.cumsum(nblk).astype(i32)])
    total = c[-1]
    j = jnp.arange(_MAXT, dtype=i32)
    ej = jnp.searchsorted(c, j, side="right").astype(i32) - 1
    ej = jnp.minimum(ej, _E - 1)
    bj = first[ej] + (j - c[ej])
    valid = j < total
    jp = total - 1
    ep = jnp.searchsorted(c, jp, side="right").astype(i32) - 1
    bp = first[ep] + (jp - c[ep])
    ej = jnp.where(valid, ej, ep)
    bj = jnp.where(valid, bj, bp)
    sj = jnp.where(valid, s_e[ej], 0)
    tj = jnp.where(valid, t_e[ej], 0)
    return bj.astype(i32), ej.astype(i32), sj.astype(i32), tj.astype(i32)


def kernel(hidden_states, Wg, W1, b1, W2, b2):
    x = hidden_states
    w_tok, e_tok = _router(x, Wg)

    iota = jnp.arange(_T, dtype=jnp.int32)
    eid_sorted, perm = lax.sort((e_tok, iota), num_keys=1)
    offsets = jnp.searchsorted(
        eid_sorted, jnp.arange(_E + 1, dtype=jnp.int32), side="left"
    ).astype(jnp.int32)
    w_sorted = w_tok[perm]
    sched_b, sched_e, sched_s, sched_t = _schedule(offsets)

    x_sorted = _sc_gather(x, perm.reshape(_NW, _NCH, _CH))
    y_sorted = _grouped_mlp(x_sorted, W1, b1, W2, b2, w_sorted,
                            sched_b, sched_e, sched_s, sched_t)
    out = _sc_scatter(y_sorted, perm.reshape(_NW, _NCH, _CH), _T)
    return out
